# baseline (speedup 1.0000x reference)
import functools

import numpy as np
import jax
import jax.numpy as jnp
from jax import lax
from jax.experimental import pallas as pl
from jax.experimental.pallas import tpu as pltpu
from jax.experimental.pallas import tpu_sc as plsc

_BINS = 30
_NC, _NS, _L = 2, 16, 16
_NW = _NC * _NS
_LUT_SIZE = 32768
_SLOTS = 32 * _L

_CHUNK = 16384


@functools.cache
def _neglog_lut() -> jax.Array:
    idx = np.arange(_LUT_SIZE, dtype=np.int64)
    bits = (idx << 15) | (1 << 14)
    vals = bits.astype(np.uint32).view(np.float32).astype(np.float64)
    with np.errstate(divide="ignore"):
        neglog = np.minimum(-np.log(vals), 100.0)
    neglog[0] = 100.0
    neglog[(0x3F800000 >> 15)] = 0.0
    return jnp.asarray(neglog.astype(np.float32))


def _sc_partials(pred_flat: jax.Array, target_flat: jax.Array) -> jax.Array:
    total = pred_flat.shape[0]
    per_w = total // _NW
    chunks = per_w // _CHUNK
    mesh = plsc.VectorSubcoreMesh(core_axis_name="c", subcore_axis_name="s")

    @functools.partial(
        pl.kernel,
        out_type=jax.ShapeDtypeStruct((2, _NW, _SLOTS), jnp.float32),
        mesh=mesh,
        compiler_params=pltpu.CompilerParams(needs_layout_passes=False),
        scratch_types=[
            pltpu.VMEM((_LUT_SIZE,), jnp.float32),
            pltpu.VMEM((_CHUNK,), jnp.float32),
            pltpu.VMEM((_CHUNK,), jnp.float32),
            pltpu.VMEM((_SLOTS,), jnp.float32),
            pltpu.VMEM((_SLOTS,), jnp.float32),
        ],
    )
    def run(pred_hbm, target_hbm, lut_hbm, out_hbm, lut_v, p_v, t_v, nb_v, sb_v):
        wid = lax.axis_index("s") * _NC + lax.axis_index("c")
        pltpu.sync_copy(lut_hbm, lut_v)

        zeros = jnp.zeros((_L,), jnp.float32)
        for j in range(_SLOTS // _L):
            nb_v[pl.ds(j * _L, _L)] = zeros
            sb_v[pl.ds(j * _L, _L)] = zeros

        lane = lax.iota(jnp.int32, _L)
        ones = jnp.ones((_L,), jnp.float32)
        elem0 = wid * per_w

        def chunk_body(c, _):
            base = elem0 + c * _CHUNK
            pltpu.sync_copy(pred_hbm.at[pl.ds(base, _CHUNK)], p_v)
            pltpu.sync_copy(target_hbm.at[pl.ds(base, _CHUNK)], t_v)

            def vec_body(i, _):
                off = i * _L
                p = p_v[pl.ds(off, _L)]
                t = t_v[pl.ds(off, _L)]
                g = jnp.abs(p - t)
                bidx = jnp.minimum((g * jnp.float32(_BINS)).astype(jnp.int32),
                                   _BINS - 1)
                bp = lax.shift_right_logical(
                    lax.bitcast_convert_type(p, jnp.int32), 15)
                q = jnp.float32(1.0) - p
                bq = lax.shift_right_logical(
                    lax.bitcast_convert_type(q, jnp.int32), 15)
                nlp = plsc.load_gather(lut_v, [bp])
                nlq = plsc.load_gather(lut_v, [bq])
                bce = t * nlp + (jnp.float32(1.0) - t) * nlq
                slot = bidx * _L + lane
                plsc.addupdate_scatter(nb_v, [slot], ones)
                plsc.addupdate_scatter(sb_v, [slot], bce)
                return _

            lax.fori_loop(0, _CHUNK // _L, vec_body, 0, unroll=4)
            return _

        lax.fori_loop(0, chunks, chunk_body, 0)

        pltpu.sync_copy(nb_v, out_hbm.at[0, wid])
        pltpu.sync_copy(sb_v, out_hbm.at[1, wid])

    return run(pred_flat, target_flat, _neglog_lut())


def _combine_body(parts_ref, out_ref):
    acc = jnp.zeros((2, _SLOTS), jnp.float32)
    for w in range(_NW):
        acc = acc + parts_ref[:, w]
    slot_bin = lax.broadcasted_iota(jnp.int32, (_SLOTS, 32), 0) // _L
    bin_col = lax.broadcasted_iota(jnp.int32, (_SLOTS, 32), 1)
    fold = (slot_bin == bin_col).astype(jnp.float32)
    per_bin = jnp.dot(acc, fold)
    counts = per_bin[0:1, :]
    sums = per_bin[1:2, :]
    nonempty = counts > 0.0
    terms = jnp.where(nonempty, sums / jnp.maximum(counts, 1.0), 0.0)
    n = jnp.sum(nonempty.astype(jnp.float32))
    loss = jnp.sum(terms) / jnp.maximum(n, 1.0)
    out_ref[:, :] = jnp.full((1, 1), loss, jnp.float32)


def kernel(pred, target, batch_size):
    del batch_size
    total = pred.shape[0] * pred.shape[1]
    pred_flat = pred.reshape(total)
    target_flat = target.astype(jnp.float32).reshape(total)

    parts = _sc_partials(pred_flat, target_flat)

    loss = pl.pallas_call(
        _combine_body,
        out_shape=jax.ShapeDtypeStruct((1, 1), jnp.float32),
    )(parts)
    return loss.reshape(())


# speedup vs baseline: 1.1353x; 1.1353x over previous
import functools

import numpy as np
import jax
import jax.numpy as jnp
from jax import lax
from jax.experimental import pallas as pl
from jax.experimental.pallas import tpu as pltpu
from jax.experimental.pallas import tpu_sc as plsc

_BINS = 30
_NC, _NS, _L = 2, 16, 16
_NW = _NC * _NS
_LUT_SIZE = 32768
_BANKS = 4
_SLOTS = 32 * _L

_CHUNK_ROWS = 16


@functools.cache
def _neglog_lut() -> jax.Array:
    idx = np.arange(_LUT_SIZE, dtype=np.int64)
    bits = (idx << 15) | (1 << 14)
    vals = bits.astype(np.uint32).view(np.float32).astype(np.float64)
    with np.errstate(divide="ignore"):
        neglog = np.minimum(-np.log(vals), 100.0)
    neglog[0] = 100.0
    neglog[(0x3F800000 >> 15)] = 0.0
    return jnp.asarray(neglog.astype(np.float32))


def _sc_partials(pred: jax.Array, target: jax.Array) -> jax.Array:
    rows, cols = pred.shape
    rows_per_w = rows // _NW
    chunks = rows_per_w // _CHUNK_ROWS
    vecs_per_row = cols // _L
    mesh = plsc.VectorSubcoreMesh(core_axis_name="c", subcore_axis_name="s")

    @functools.partial(
        pl.kernel,
        out_type=jax.ShapeDtypeStruct((2, _NW, _BANKS * _SLOTS), jnp.float32),
        mesh=mesh,
        compiler_params=pltpu.CompilerParams(needs_layout_passes=False),
        scratch_types=[
            pltpu.VMEM((_LUT_SIZE,), jnp.float32),
            pltpu.VMEM((_CHUNK_ROWS, cols), jnp.float32),
            pltpu.VMEM((_CHUNK_ROWS, cols), jnp.float32),
            pltpu.VMEM((_BANKS * _SLOTS,), jnp.float32),
            pltpu.VMEM((_BANKS * _SLOTS,), jnp.float32),
        ],
    )
    def run(pred_hbm, target_hbm, lut_hbm, out_hbm, lut_v, p_v, t_v, nb_v, sb_v):
        wid = lax.axis_index("s") * _NC + lax.axis_index("c")
        pltpu.sync_copy(lut_hbm, lut_v)

        zeros = jnp.zeros((_L,), jnp.float32)
        for j in range(_BANKS * _SLOTS // _L):
            nb_v[pl.ds(j * _L, _L)] = zeros
            sb_v[pl.ds(j * _L, _L)] = zeros

        lane = lax.iota(jnp.int32, _L)
        lanes = [lane + jnp.int32(k * _SLOTS) for k in range(_BANKS)]
        ones = jnp.ones((_L,), jnp.float32)
        row0 = wid * rows_per_w

        def chunk_body(c, _):
            base = row0 + c * _CHUNK_ROWS
            pltpu.sync_copy(pred_hbm.at[pl.ds(base, _CHUNK_ROWS)], p_v)
            pltpu.sync_copy(target_hbm.at[pl.ds(base, _CHUNK_ROWS)], t_v)

            def row_body(i, _):
                def quad_body(j, _):
                    col0 = j * (_BANKS * _L)
                    for k in range(_BANKS):
                        off = col0 + k * _L
                        p = p_v[i, pl.ds(off, _L)]
                        t = t_v[i, pl.ds(off, _L)]
                        g = jnp.abs(p - t)
                        bidx = jnp.minimum(
                            (g * jnp.float32(_BINS)).astype(jnp.int32),
                            _BINS - 1)
                        bp = lax.shift_right_logical(
                            lax.bitcast_convert_type(p, jnp.int32), 15)
                        q = jnp.float32(1.0) - p
                        bq = lax.shift_right_logical(
                            lax.bitcast_convert_type(q, jnp.int32), 15)
                        nlp = plsc.load_gather(lut_v, [bp])
                        nlq = plsc.load_gather(lut_v, [bq])
                        bce = t * nlp + (jnp.float32(1.0) - t) * nlq
                        slot = bidx * _L + lanes[k]
                        plsc.addupdate_scatter(nb_v, [slot], ones)
                        plsc.addupdate_scatter(sb_v, [slot], bce)
                    return _

                lax.fori_loop(0, vecs_per_row // _BANKS, quad_body, 0)
                return _

            lax.fori_loop(0, _CHUNK_ROWS, row_body, 0)
            return _

        lax.fori_loop(0, chunks, chunk_body, 0)

        pltpu.sync_copy(nb_v, out_hbm.at[0, wid])
        pltpu.sync_copy(sb_v, out_hbm.at[1, wid])

    return run(pred, target, _neglog_lut())


def _combine_body(parts_ref, out_ref):
    nslots = _BANKS * _SLOTS
    acc = jnp.zeros((2, nslots), jnp.float32)
    for w in range(_NW):
        acc = acc + parts_ref[:, w]
    slot_bin = (lax.broadcasted_iota(jnp.int32, (nslots, 32), 0) % _SLOTS) // _L
    bin_col = lax.broadcasted_iota(jnp.int32, (nslots, 32), 1)
    fold = (slot_bin == bin_col).astype(jnp.float32)
    per_bin = jnp.dot(acc, fold)
    counts = per_bin[0:1, :]
    sums = per_bin[1:2, :]
    nonempty = counts > 0.0
    terms = jnp.where(nonempty, sums / jnp.maximum(counts, 1.0), 0.0)
    n = jnp.sum(nonempty.astype(jnp.float32))
    loss = jnp.sum(terms) / jnp.maximum(n, 1.0)
    out_ref[:, :] = jnp.full((1, 1), loss, jnp.float32)


def kernel(pred, target, batch_size):
    del batch_size
    parts = _sc_partials(pred, target.astype(jnp.float32))

    loss = pl.pallas_call(
        _combine_body,
        out_shape=jax.ShapeDtypeStruct((1, 1), jnp.float32),
    )(parts)
    return loss.reshape(())


# speedup vs baseline: 2.6221x; 2.3097x over previous
import functools

import numpy as np
import jax
import jax.numpy as jnp
from jax import lax
from jax.experimental import pallas as pl
from jax.experimental.pallas import tpu as pltpu
from jax.experimental.pallas import tpu_sc as plsc

_BINS = 30
_NC, _NS, _L = 2, 16, 16
_NW = _NC * _NS
_LUT_SIZE = 32768
_BANKS = 4
_SLOTS = 32 * _L

_CHUNK_ROWS = 16


@functools.cache
def _neglog_lut() -> jax.Array:
    idx = np.arange(_LUT_SIZE, dtype=np.int64)
    bits = (idx << 15) | (1 << 14)
    vals = bits.astype(np.uint32).view(np.float32).astype(np.float64)
    with np.errstate(divide="ignore"):
        neglog = np.minimum(-np.log(vals), 100.0)
    neglog[0] = 100.0
    neglog[(0x3F800000 >> 15)] = 0.0
    return jnp.asarray(neglog.astype(np.float32))


def _sc_partials(pred: jax.Array, target: jax.Array) -> jax.Array:
    rows, cols = pred.shape
    rows_per_w = rows // _NW
    chunks = rows_per_w // _CHUNK_ROWS
    vecs_per_row = cols // _L
    mesh = plsc.VectorSubcoreMesh(core_axis_name="c", subcore_axis_name="s")

    @functools.partial(
        pl.kernel,
        out_type=jax.ShapeDtypeStruct((2, _NW, _BANKS * _SLOTS), jnp.float32),
        mesh=mesh,
        compiler_params=pltpu.CompilerParams(needs_layout_passes=False),
        scratch_types=[
            pltpu.VMEM((_LUT_SIZE,), jnp.float32),
            pltpu.VMEM((_CHUNK_ROWS, cols), jnp.float32),
            pltpu.VMEM((_CHUNK_ROWS, cols), jnp.float32),
            pltpu.VMEM((_BANKS * _SLOTS,), jnp.float32),
            pltpu.VMEM((_BANKS * _SLOTS,), jnp.float32),
        ],
    )
    def run(pred_hbm, target_hbm, lut_hbm, out_hbm, lut_v, p_v, t_v, nb_v, sb_v):
        wid = lax.axis_index("s") * _NC + lax.axis_index("c")
        pltpu.sync_copy(lut_hbm, lut_v)

        zeros = jnp.zeros((_L,), jnp.float32)
        for j in range(_BANKS * _SLOTS // _L):
            nb_v[pl.ds(j * _L, _L)] = zeros
            sb_v[pl.ds(j * _L, _L)] = zeros

        lane = lax.iota(jnp.int32, _L)
        lanes = [lane + jnp.int32(k * _SLOTS) for k in range(_BANKS)]
        ones = jnp.ones((_L,), jnp.float32)
        row0 = wid * rows_per_w

        def chunk_body(c, _):
            base = row0 + c * _CHUNK_ROWS
            pltpu.sync_copy(pred_hbm.at[pl.ds(base, _CHUNK_ROWS)], p_v)
            pltpu.sync_copy(target_hbm.at[pl.ds(base, _CHUNK_ROWS)], t_v)

            def row_body(i, _):
                @plsc.parallel_loop(0, vecs_per_row // _BANKS)
                def quad_body(j):
                    col0 = j * (_BANKS * _L)
                    for k in range(_BANKS):
                        off = col0 + k * _L
                        p = p_v[i, pl.ds(off, _L)]
                        t = t_v[i, pl.ds(off, _L)]
                        g = jnp.abs(p - t)
                        bidx = jnp.minimum(
                            (g * jnp.float32(_BINS)).astype(jnp.int32),
                            _BINS - 1)
                        bp = lax.shift_right_logical(
                            lax.bitcast_convert_type(p, jnp.int32), 15)
                        q = jnp.float32(1.0) - p
                        bq = lax.shift_right_logical(
                            lax.bitcast_convert_type(q, jnp.int32), 15)
                        nlp = plsc.load_gather(lut_v, [bp])
                        nlq = plsc.load_gather(lut_v, [bq])
                        bce = t * nlp + (jnp.float32(1.0) - t) * nlq
                        slot = bidx * _L + lanes[k]
                        plsc.addupdate_scatter(nb_v, [slot], ones)
                        plsc.addupdate_scatter(sb_v, [slot], bce)
                return _

            lax.fori_loop(0, _CHUNK_ROWS, row_body, 0)
            return _

        lax.fori_loop(0, chunks, chunk_body, 0)

        pltpu.sync_copy(nb_v, out_hbm.at[0, wid])
        pltpu.sync_copy(sb_v, out_hbm.at[1, wid])

    return run(pred, target, _neglog_lut())


def _combine_body(parts_ref, out_ref):
    nslots = _BANKS * _SLOTS
    acc = jnp.zeros((2, nslots), jnp.float32)
    for w in range(_NW):
        acc = acc + parts_ref[:, w]
    slot_bin = (lax.broadcasted_iota(jnp.int32, (nslots, 32), 0) % _SLOTS) // _L
    bin_col = lax.broadcasted_iota(jnp.int32, (nslots, 32), 1)
    fold = (slot_bin == bin_col).astype(jnp.float32)
    per_bin = jnp.dot(acc, fold)
    counts = per_bin[0:1, :]
    sums = per_bin[1:2, :]
    nonempty = counts > 0.0
    terms = jnp.where(nonempty, sums / jnp.maximum(counts, 1.0), 0.0)
    n = jnp.sum(nonempty.astype(jnp.float32))
    loss = jnp.sum(terms) / jnp.maximum(n, 1.0)
    out_ref[:, :] = jnp.full((1, 1), loss, jnp.float32)


def kernel(pred, target, batch_size):
    del batch_size
    parts = _sc_partials(pred, target.astype(jnp.float32))

    loss = pl.pallas_call(
        _combine_body,
        out_shape=jax.ShapeDtypeStruct((1, 1), jnp.float32),
    )(parts)
    return loss.reshape(())


# speedup vs baseline: 4.1444x; 1.5806x over previous
import functools

import numpy as np
import jax
import jax.numpy as jnp
from jax import lax
from jax.experimental import pallas as pl
from jax.experimental.pallas import tpu as pltpu
from jax.experimental.pallas import tpu_sc as plsc

_BINS = 30
_NC, _NS, _L = 2, 16, 16
_NW = _NC * _NS
_LUT_SIZE = 32768
_BANKS = 4
_SLOTS = 32 * _L

_CHUNK_ROWS = 16


@functools.cache
def _neglog_lut() -> jax.Array:
    idx = np.arange(_LUT_SIZE, dtype=np.int64)
    bits = (idx << 15) | (1 << 14)
    vals = bits.astype(np.uint32).view(np.float32).astype(np.float64)
    with np.errstate(divide="ignore"):
        neglog = np.minimum(-np.log(vals), 100.0)
    neglog[0] = 100.0
    neglog[(0x3F800000 >> 15)] = 0.0
    return jnp.asarray(neglog.astype(np.float32))


def _sc_partials(pred: jax.Array, target: jax.Array) -> jax.Array:
    rows, cols = pred.shape
    rows_per_w = rows // _NW
    chunks = rows_per_w // _CHUNK_ROWS
    vecs_per_row = cols // _L
    quads = _CHUNK_ROWS * vecs_per_row // _BANKS
    quads_per_row = vecs_per_row // _BANKS
    mesh = plsc.VectorSubcoreMesh(core_axis_name="c", subcore_axis_name="s")

    @functools.partial(
        pl.kernel,
        out_type=jax.ShapeDtypeStruct((2, _NW, _BANKS * _SLOTS), jnp.float32),
        mesh=mesh,
        compiler_params=pltpu.CompilerParams(needs_layout_passes=False),
        scratch_types=[
            pltpu.VMEM((_LUT_SIZE,), jnp.float32),
            pltpu.VMEM((2, _CHUNK_ROWS, cols), jnp.float32),
            pltpu.VMEM((2, _CHUNK_ROWS, cols), jnp.float32),
            pltpu.VMEM((_BANKS * _SLOTS,), jnp.float32),
            pltpu.VMEM((_BANKS * _SLOTS,), jnp.float32),
            pltpu.SemaphoreType.DMA,
            pltpu.SemaphoreType.DMA,
        ],
    )
    def run(pred_hbm, target_hbm, lut_hbm, out_hbm,
            lut_v, p_v, t_v, nb_v, sb_v, sem0, sem1):
        wid = lax.axis_index("s") * _NC + lax.axis_index("c")
        pltpu.sync_copy(lut_hbm, lut_v)

        zeros = jnp.zeros((_L,), jnp.float32)
        for j in range(_BANKS * _SLOTS // _L):
            nb_v[pl.ds(j * _L, _L)] = zeros
            sb_v[pl.ds(j * _L, _L)] = zeros

        lane = lax.iota(jnp.int32, _L)
        lanes = [lane + jnp.int32(k * _SLOTS) for k in range(_BANKS)]
        ones = jnp.ones((_L,), jnp.float32)
        row0 = wid * rows_per_w
        sems = (sem0, sem1)

        def copies(c, b):
            base = row0 + c * _CHUNK_ROWS
            return (
                pltpu.make_async_copy(
                    pred_hbm.at[pl.ds(base, _CHUNK_ROWS)], p_v.at[b], sems[b]),
                pltpu.make_async_copy(
                    target_hbm.at[pl.ds(base, _CHUNK_ROWS)], t_v.at[b], sems[b]),
            )

        def start(c, b):
            for cp in copies(c, b):
                cp.start()

        def wait(c, b):
            for cp in copies(c, b):
                cp.wait()

        def process(b):
            @plsc.parallel_loop(0, quads)
            def quad_body(v):
                i = lax.shift_right_logical(v, 4)
                col0 = (v & (quads_per_row - 1)) * (_BANKS * _L)
                for k in range(_BANKS):
                    off = col0 + k * _L
                    p = p_v[b, i, pl.ds(off, _L)]
                    t = t_v[b, i, pl.ds(off, _L)]
                    g = jnp.abs(p - t)
                    bidx = jnp.minimum(
                        (g * jnp.float32(_BINS)).astype(jnp.int32),
                        _BINS - 1)
                    bp = lax.shift_right_logical(
                        lax.bitcast_convert_type(p, jnp.int32), 15)
                    q = jnp.float32(1.0) - p
                    bq = lax.shift_right_logical(
                        lax.bitcast_convert_type(q, jnp.int32), 15)
                    nlp = plsc.load_gather(lut_v, [bp])
                    nlq = plsc.load_gather(lut_v, [bq])
                    bce = t * nlp + (jnp.float32(1.0) - t) * nlq
                    slot = bidx * _L + lanes[k]
                    plsc.addupdate_scatter(nb_v, [slot], ones)
                    plsc.addupdate_scatter(sb_v, [slot], bce)

        start(0, 0)

        def pair_body(j, _):
            c0 = 2 * j
            start(c0 + 1, 1)
            wait(c0, 0)
            process(0)

            @pl.when(j < chunks // 2 - 1)
            def _start_next():
                start(c0 + 2, 0)

            wait(c0 + 1, 1)
            process(1)
            return _

        lax.fori_loop(0, chunks // 2, pair_body, 0)

        pltpu.sync_copy(nb_v, out_hbm.at[0, wid])
        pltpu.sync_copy(sb_v, out_hbm.at[1, wid])

    return run(pred, target, _neglog_lut())


def _combine_body(parts_ref, out_ref):
    nslots = _BANKS * _SLOTS
    acc = jnp.zeros((2, nslots), jnp.float32)
    for w in range(_NW):
        acc = acc + parts_ref[:, w]
    slot_bin = (lax.broadcasted_iota(jnp.int32, (nslots, 32), 0) % _SLOTS) // _L
    bin_col = lax.broadcasted_iota(jnp.int32, (nslots, 32), 1)
    fold = (slot_bin == bin_col).astype(jnp.float32)
    per_bin = jnp.dot(acc, fold)
    counts = per_bin[0:1, :]
    sums = per_bin[1:2, :]
    nonempty = counts > 0.0
    terms = jnp.where(nonempty, sums / jnp.maximum(counts, 1.0), 0.0)
    n = jnp.sum(nonempty.astype(jnp.float32))
    loss = jnp.sum(terms) / jnp.maximum(n, 1.0)
    out_ref[:, :] = jnp.full((1, 1), loss, jnp.float32)


def kernel(pred, target, batch_size):
    del batch_size
    parts = _sc_partials(pred, target.astype(jnp.float32))

    loss = pl.pallas_call(
        _combine_body,
        out_shape=jax.ShapeDtypeStruct((1, 1), jnp.float32),
    )(parts)
    return loss.reshape(())
